# baseline (speedup 1.0000x reference)
import functools

import jax
import jax.numpy as jnp
from jax import lax
from jax.experimental import pallas as pl
from jax.experimental.pallas import tpu as pltpu
from jax.experimental.pallas import tpu_sc as plsc

V = 10000
VP = 10240
E = 320000
FIN = 128
FOUT = 128
K = 3

NC = 2
NS = 16
LANES = 16
EPT = E // (NC * NS)
C = 80
NCH = EPT // C
RPT = VP // NS
NZ = RPT // C


def _spmm_body(rows_hbm, cols_hbm, vals_hbm, x_hbm,
               ya_hbm, yb_hbm,
               vals_v, cols_all, r0, r1, g0, g1, acc,
               sem_g0, sem_g1, sem_r0, sem_r1, ss0, ss1):
    c = lax.axis_index("c")
    s = lax.axis_index("s")

    zeros16 = jnp.zeros((LANES,), jnp.float32)
    for r in range(C):
        for j in range(FIN // LANES):
            g0[r, pl.ds(j * LANES, LANES)] = zeros16
            g1[r, pl.ds(j * LANES, LANES)] = zeros16

    def zinit(k, carry):
        pltpu.sync_copy(g0, acc.at[pl.ds(s * RPT + k * C, C)])
        return carry
    lax.fori_loop(0, NZ, zinit, 0)

    ebase = (c * NS + s) * EPT
    pltpu.sync_copy(vals_hbm.at[pl.ds(ebase, EPT)], vals_v.at[pl.ds(0, EPT)])
    pltpu.sync_copy(cols_hbm.at[pl.ds(ebase, EPT)], cols_all)

    plsc.subcore_barrier()

    def scale(k, g):
        base = (k * C).astype(jnp.int32)
        for go in range(0, C, LANES):
            vv16 = vals_v[pl.ds(base + go, LANES)]
            for ei in range(min(LANES, C - go)):
                e = go + ei
                lane = jnp.full((LANES, 1), ei, jnp.int32)
                vv = lax.gather(
                    vv16, lane,
                    lax.GatherDimensionNumbers(
                        offset_dims=(), collapsed_slice_dims=(0,),
                        start_index_map=(0,)),
                    slice_sizes=(1,),
                    mode=lax.GatherScatterMode.PROMISE_IN_BOUNDS)
                for j in range(FIN // LANES):
                    sl = pl.ds(j * LANES, LANES)
                    g[e, sl] = g[e, sl] * vv

    def start_gather(k, g, sg):
        pltpu.async_copy(x_hbm.at[cols_all.at[pl.ds(k * C, C)]], g, sg)

    def start_rows(k, r, sr):
        pltpu.async_copy(rows_hbm.at[pl.ds(ebase + k * C, C)], r, sr)

    def wait_gather(g, sg):
        pltpu.make_async_copy(x_hbm.at[cols_all.at[pl.ds(0, C)]], g, sg).wait()

    def wait_rows(r, sr):
        pltpu.make_async_copy(rows_hbm.at[pl.ds(ebase, C)], r, sr).wait()

    def start_scatter(scl, r, ss):
        pltpu.async_copy(scl, acc.at[r], ss, add=True)

    def wait_scatter(scl, r, ss):
        pltpu.make_async_copy(scl, acc.at[r], ss).wait()

    last = jnp.int32(NCH - 1)

    pltpu.sync_copy(rows_hbm.at[pl.ds(ebase, C)], r1)
    start_scatter(g1, r1, ss1)
    start_gather(jnp.int32(0), g0, sem_g0)
    start_rows(jnp.int32(0), r0, sem_r0)

    def pair(i, carry):
        k0 = (2 * i).astype(jnp.int32)
        wait_gather(g0, sem_g0)
        scale(k0, g0)
        wait_scatter(g1, r1, ss1)
        start_gather(k0 + 1, g1, sem_g1)
        start_rows(k0 + 1, r1, sem_r1)
        wait_rows(r0, sem_r0)
        start_scatter(g0, r0, ss0)
        wait_gather(g1, sem_g1)
        scale(k0 + 1, g1)
        wait_scatter(g0, r0, ss0)
        start_gather(jnp.minimum(k0 + 2, last), g0, sem_g0)
        start_rows(jnp.minimum(k0 + 2, last), r0, sem_r0)
        wait_rows(r1, sem_r1)
        start_scatter(g1, r1, ss1)
        return carry
    lax.fori_loop(0, NCH // 2, pair, 0)

    wait_gather(g0, sem_g0)
    scale(last, g0)
    wait_scatter(g1, r1, ss1)
    wait_rows(r0, sem_r0)
    start_scatter(g0, r0, ss0)
    wait_scatter(g0, r0, ss0)

    plsc.subcore_barrier()

    rb = s * RPT

    @pl.when(c == 0)
    def _():
        pltpu.sync_copy(acc.at[pl.ds(rb, RPT)], ya_hbm.at[pl.ds(rb, RPT)])

    @pl.when(c == 1)
    def _():
        pltpu.sync_copy(acc.at[pl.ds(rb, RPT)], yb_hbm.at[pl.ds(rb, RPT)])


_spmm_sc = functools.partial(
    pl.kernel,
    out_type=(jax.ShapeDtypeStruct((VP, FIN), jnp.float32),
              jax.ShapeDtypeStruct((VP, FIN), jnp.float32)),
    mesh=plsc.VectorSubcoreMesh(core_axis_name="c", subcore_axis_name="s",
                                num_cores=NC, num_subcores=NS),
    scratch_types=[
        pltpu.VMEM((EPT + LANES,), jnp.float32),
        pltpu.VMEM((EPT,), jnp.int32),
        pltpu.VMEM((C,), jnp.int32),
        pltpu.VMEM((C,), jnp.int32),
        pltpu.VMEM((C, FIN), jnp.float32),
        pltpu.VMEM((C, FIN), jnp.float32),
        pltpu.VMEM_SHARED((VP, FIN), jnp.float32),
        pltpu.SemaphoreType.DMA,
        pltpu.SemaphoreType.DMA,
        pltpu.SemaphoreType.DMA,
        pltpu.SemaphoreType.DMA,
        pltpu.SemaphoreType.DMA,
        pltpu.SemaphoreType.DMA,
    ],
    compiler_params=pltpu.CompilerParams(needs_layout_passes=False),
)(_spmm_body)


_ROWS_BLK = 1024


def _add_body(a_ref, b_ref, o_ref):
    o_ref[...] = a_ref[...] + b_ref[...]


def _combine(a, b):
    return pl.pallas_call(
        _add_body,
        grid=(VP // _ROWS_BLK,),
        in_specs=[
            pl.BlockSpec((_ROWS_BLK, FIN), lambda i: (i, 0)),
            pl.BlockSpec((_ROWS_BLK, FIN), lambda i: (i, 0)),
        ],
        out_specs=pl.BlockSpec((_ROWS_BLK, FIN), lambda i: (i, 0)),
        out_shape=jax.ShapeDtypeStruct((VP, FIN), jnp.float32),
    )(a, b)


def _matmul_body(x0_ref, x1_ref, ta_ref, tb_ref, w_ref, b_ref, out_ref):
    w0 = w_ref[0]
    w1 = w_ref[1]
    w2 = w_ref[2]
    acc = jnp.dot(x0_ref[...], w0 - w2, preferred_element_type=jnp.float32)
    acc = acc + jnp.dot(x1_ref[...], w1, preferred_element_type=jnp.float32)
    xt = ta_ref[...] + tb_ref[...]
    acc = acc + jnp.dot(xt, w2 + w2, preferred_element_type=jnp.float32)
    out_ref[...] = acc + b_ref[...]


def _cheb_matmul(x0, x1, ta, tb, wt, bias2d):
    grid = (VP // _ROWS_BLK,)
    return pl.pallas_call(
        _matmul_body,
        grid=grid,
        in_specs=[
            pl.BlockSpec((_ROWS_BLK, FIN), lambda i: (i, 0)),
            pl.BlockSpec((_ROWS_BLK, FIN), lambda i: (i, 0)),
            pl.BlockSpec((_ROWS_BLK, FIN), lambda i: (i, 0)),
            pl.BlockSpec((_ROWS_BLK, FIN), lambda i: (i, 0)),
            pl.BlockSpec((K, FIN, FOUT), lambda i: (0, 0, 0)),
            pl.BlockSpec((1, FOUT), lambda i: (0, 0)),
        ],
        out_specs=pl.BlockSpec((_ROWS_BLK, FOUT), lambda i: (i, 0)),
        out_shape=jax.ShapeDtypeStruct((VP, FOUT), jnp.float32),
    )(x0, x1, ta, tb, wt, bias2d)


def kernel(lap_indices, lap_values, inputs, weight, bias):
    rows = lap_indices[0]
    cols = lap_indices[1]
    x0 = jnp.pad(inputs.reshape(V, FIN), ((0, VP - V), (0, 0)))

    y_a, y_b = _spmm_sc(rows, cols, lap_values, x0)
    x1 = _combine(y_a, y_b)
    t_a, t_b = _spmm_sc(rows, cols, lap_values, x1)

    wt = jnp.transpose(weight, (1, 0, 2))
    out = _cheb_matmul(x0, x1, t_a, t_b, wt, bias.reshape(1, FOUT))
    return out[:V].reshape(1, V, FOUT)


# speedup vs baseline: 1.3055x; 1.3055x over previous
import functools

import jax
import jax.numpy as jnp
from jax import lax
from jax.experimental import pallas as pl
from jax.experimental.pallas import tpu as pltpu
from jax.experimental.pallas import tpu_sc as plsc

V = 10000
VP = 10240
E = 320000
FIN = 128
FOUT = 128
K = 3

NC = 2
NS = 16
LANES = 16
EPT = E // (NC * NS)
C = 40
NCH = EPT // C
RPT = VP // NS
NZ = RPT // C


def _spmm_body(rows_hbm, cols_hbm, vals_hbm, x_hbm,
               ya_hbm, yb_hbm,
               vals_v, cols_all, r0, r1, g0, g1, scl0, scl1, acc,
               sem_g0, sem_g1, sem_r0, sem_r1, ss0, ss1):
    c = lax.axis_index("c")
    s = lax.axis_index("s")

    zeros16 = jnp.zeros((LANES,), jnp.float32)
    for r in range(C):
        for j in range(FIN // LANES):
            scl0[r, pl.ds(j * LANES, LANES)] = zeros16
            scl1[r, pl.ds(j * LANES, LANES)] = zeros16

    def zinit(k, carry):
        pltpu.sync_copy(scl0, acc.at[pl.ds(s * RPT + k * C, C)])
        return carry
    lax.fori_loop(0, NZ, zinit, 0)

    ebase = (c * NS + s) * EPT
    pltpu.sync_copy(vals_hbm.at[pl.ds(ebase, EPT)], vals_v.at[pl.ds(0, EPT)])
    pltpu.sync_copy(cols_hbm.at[pl.ds(ebase, EPT)], cols_all)

    plsc.subcore_barrier()

    def scale(k, g, scl):
        base = (k * C).astype(jnp.int32)
        for go in range(0, C, LANES):
            vv16 = vals_v[pl.ds(base + go, LANES)]
            for ei in range(min(LANES, C - go)):
                e = go + ei
                lane = jnp.full((LANES, 1), ei, jnp.int32)
                vv = lax.gather(
                    vv16, lane,
                    lax.GatherDimensionNumbers(
                        offset_dims=(), collapsed_slice_dims=(0,),
                        start_index_map=(0,)),
                    slice_sizes=(1,),
                    mode=lax.GatherScatterMode.PROMISE_IN_BOUNDS)
                for j in range(FIN // LANES):
                    sl = pl.ds(j * LANES, LANES)
                    scl[e, sl] = g[e, sl] * vv

    def start_gather(k, g, sg):
        pltpu.async_copy(x_hbm.at[cols_all.at[pl.ds(k * C, C)]], g, sg)

    def start_rows(k, r, sr):
        pltpu.async_copy(rows_hbm.at[pl.ds(ebase + k * C, C)], r, sr)

    def wait_gather(g, sg):
        pltpu.make_async_copy(x_hbm.at[cols_all.at[pl.ds(0, C)]], g, sg).wait()

    def wait_rows(r, sr):
        pltpu.make_async_copy(rows_hbm.at[pl.ds(ebase, C)], r, sr).wait()

    def start_scatter(scl, r, ss):
        pltpu.async_copy(scl, acc.at[r], ss, add=True)

    def wait_scatter(scl, r, ss):
        pltpu.make_async_copy(scl, acc.at[r], ss).wait()

    last = jnp.int32(NCH - 1)

    start_gather(jnp.int32(0), g0, sem_g0)
    start_gather(jnp.int32(1), g1, sem_g1)
    pltpu.sync_copy(rows_hbm.at[pl.ds(ebase, C)], r1)
    start_rows(jnp.int32(0), r0, sem_r0)
    start_scatter(scl1, r1, ss1)

    def pair(i, carry):
        k0 = (2 * i).astype(jnp.int32)
        wait_gather(g0, sem_g0)
        scale(k0, g0, scl0)
        start_gather(jnp.minimum(k0 + 2, last), g0, sem_g0)
        wait_scatter(scl1, r1, ss1)
        start_rows(k0 + 1, r1, sem_r1)
        wait_rows(r0, sem_r0)
        start_scatter(scl0, r0, ss0)
        wait_gather(g1, sem_g1)
        scale(k0 + 1, g1, scl1)
        start_gather(jnp.minimum(k0 + 3, last), g1, sem_g1)
        wait_scatter(scl0, r0, ss0)
        start_rows(jnp.minimum(k0 + 2, last), r0, sem_r0)
        wait_rows(r1, sem_r1)
        start_scatter(scl1, r1, ss1)
        return carry
    lax.fori_loop(0, NCH // 2, pair, 0)

    wait_scatter(scl1, r1, ss1)
    wait_rows(r0, sem_r0)
    wait_gather(g0, sem_g0)
    wait_gather(g1, sem_g1)

    plsc.subcore_barrier()

    rb = s * RPT

    @pl.when(c == 0)
    def _():
        pltpu.sync_copy(acc.at[pl.ds(rb, RPT)], ya_hbm.at[pl.ds(rb, RPT)])

    @pl.when(c == 1)
    def _():
        pltpu.sync_copy(acc.at[pl.ds(rb, RPT)], yb_hbm.at[pl.ds(rb, RPT)])


_spmm_sc = functools.partial(
    pl.kernel,
    out_type=(jax.ShapeDtypeStruct((VP, FIN), jnp.float32),
              jax.ShapeDtypeStruct((VP, FIN), jnp.float32)),
    mesh=plsc.VectorSubcoreMesh(core_axis_name="c", subcore_axis_name="s",
                                num_cores=NC, num_subcores=NS),
    scratch_types=[
        pltpu.VMEM((EPT + LANES,), jnp.float32),
        pltpu.VMEM((EPT,), jnp.int32),
        pltpu.VMEM((C,), jnp.int32),
        pltpu.VMEM((C,), jnp.int32),
        pltpu.VMEM((C, FIN), jnp.float32),
        pltpu.VMEM((C, FIN), jnp.float32),
        pltpu.VMEM((C, FIN), jnp.float32),
        pltpu.VMEM((C, FIN), jnp.float32),
        pltpu.VMEM_SHARED((VP, FIN), jnp.float32),
        pltpu.SemaphoreType.DMA,
        pltpu.SemaphoreType.DMA,
        pltpu.SemaphoreType.DMA,
        pltpu.SemaphoreType.DMA,
        pltpu.SemaphoreType.DMA,
        pltpu.SemaphoreType.DMA,
    ],
    compiler_params=pltpu.CompilerParams(needs_layout_passes=False),
)(_spmm_body)


_ROWS_BLK = 1024


def _add_body(a_ref, b_ref, o_ref):
    o_ref[...] = a_ref[...] + b_ref[...]


def _combine(a, b):
    return pl.pallas_call(
        _add_body,
        grid=(VP // _ROWS_BLK,),
        in_specs=[
            pl.BlockSpec((_ROWS_BLK, FIN), lambda i: (i, 0)),
            pl.BlockSpec((_ROWS_BLK, FIN), lambda i: (i, 0)),
        ],
        out_specs=pl.BlockSpec((_ROWS_BLK, FIN), lambda i: (i, 0)),
        out_shape=jax.ShapeDtypeStruct((VP, FIN), jnp.float32),
    )(a, b)


def _matmul_body(x0_ref, x1_ref, ta_ref, tb_ref, w_ref, b_ref, out_ref):
    w0 = w_ref[0]
    w1 = w_ref[1]
    w2 = w_ref[2]
    acc = jnp.dot(x0_ref[...], w0 - w2, preferred_element_type=jnp.float32)
    acc = acc + jnp.dot(x1_ref[...], w1, preferred_element_type=jnp.float32)
    xt = ta_ref[...] + tb_ref[...]
    acc = acc + jnp.dot(xt, w2 + w2, preferred_element_type=jnp.float32)
    out_ref[...] = acc + b_ref[...]


def _cheb_matmul(x0, x1, ta, tb, wt, bias2d):
    grid = (VP // _ROWS_BLK,)
    return pl.pallas_call(
        _matmul_body,
        grid=grid,
        in_specs=[
            pl.BlockSpec((_ROWS_BLK, FIN), lambda i: (i, 0)),
            pl.BlockSpec((_ROWS_BLK, FIN), lambda i: (i, 0)),
            pl.BlockSpec((_ROWS_BLK, FIN), lambda i: (i, 0)),
            pl.BlockSpec((_ROWS_BLK, FIN), lambda i: (i, 0)),
            pl.BlockSpec((K, FIN, FOUT), lambda i: (0, 0, 0)),
            pl.BlockSpec((1, FOUT), lambda i: (0, 0)),
        ],
        out_specs=pl.BlockSpec((_ROWS_BLK, FOUT), lambda i: (i, 0)),
        out_shape=jax.ShapeDtypeStruct((VP, FOUT), jnp.float32),
    )(x0, x1, ta, tb, wt, bias2d)


def kernel(lap_indices, lap_values, inputs, weight, bias):
    rows = lap_indices[0]
    cols = lap_indices[1]
    x0 = jnp.pad(inputs.reshape(V, FIN), ((0, VP - V), (0, 0)))

    y_a, y_b = _spmm_sc(rows, cols, lap_values, x0)
    x1 = _combine(y_a, y_b)
    t_a, t_b = _spmm_sc(rows, cols, lap_values, x1)

    wt = jnp.transpose(weight, (1, 0, 2))
    out = _cheb_matmul(x0, x1, t_a, t_b, wt, bias.reshape(1, FOUT))
    return out[:V].reshape(1, V, FOUT)
